# speedup vs baseline: 9.1838x; 9.1838x over previous
import functools

import jax
import jax.numpy as jnp
from jax import lax
from jax.experimental import pallas as pl
from jax.experimental.pallas import tpu as pltpu
from jax.experimental.pallas import tpu_sc as plsc

D = 128
NC, NS = 2, 16
NW = NC * NS
B = 4096 * 200
BPW = B // NW
GSZ = 128
CH = 256
NSUB = CH // GSZ
NCHUNK = BPW // CH
NROWS_W = BPW // GSZ

_mesh = plsc.VectorSubcoreMesh(core_axis_name="c", subcore_axis_name="s")


@functools.partial(
    pl.kernel,
    out_type=jax.ShapeDtypeStruct((B, D), jnp.float32),
    mesh=_mesh,
    scratch_types=[
        pltpu.VMEM((NROWS_W, GSZ), jnp.int32),
        pltpu.VMEM((2, CH, D), jnp.float32),
        pltpu.SemaphoreType.DMA,
        pltpu.SemaphoreType.DMA,
        pltpu.SemaphoreType.DMA,
        pltpu.SemaphoreType.DMA,
    ],
)
def _sc_gather(q_hbm, table_hbm, out_hbm, idx_v, rows_v, g0, g1, o0, o1):
    wid = lax.axis_index("s") * NC + lax.axis_index("c")
    row0 = wid * NROWS_W
    base = wid * BPW
    gsem = (g0, g1)
    osem = (o0, o1)

    pltpu.sync_copy(q_hbm.at[pl.ds(row0, NROWS_W), :], idx_v)

    def fire_gathers(c, buf):
        for j in range(NSUB):
            pltpu.async_copy(
                table_hbm.at[idx_v.at[c * NSUB + j]],
                rows_v.at[buf, pl.ds(j * GSZ, GSZ), :],
                gsem[buf],
            )

    def wait_gathers(c, buf):
        for j in range(NSUB):
            pltpu.make_async_copy(
                table_hbm.at[idx_v.at[c * NSUB + j]],
                rows_v.at[buf, pl.ds(j * GSZ, GSZ), :],
                gsem[buf],
            ).wait()

    def fire_out(c, buf):
        pltpu.async_copy(
            rows_v.at[buf],
            out_hbm.at[pl.ds(base + c * CH, CH), :],
            osem[buf],
        )

    def wait_out(c, buf):
        pltpu.make_async_copy(
            rows_v.at[buf],
            out_hbm.at[pl.ds(base + c * CH, CH), :],
            osem[buf],
        ).wait()

    fire_gathers(0, 0)
    fire_gathers(1, 1)

    def step(c, buf):
        wait_gathers(c - 2, buf)
        fire_out(c - 2, buf)
        wait_out(c - 2, buf)
        fire_gathers(c, buf)

    def pair(p, carry):
        step(2 * p, 0)
        step(2 * p + 1, 1)
        return carry

    lax.fori_loop(1, NCHUNK // 2, pair, 0)

    for c, buf in ((NCHUNK - 2, 0), (NCHUNK - 1, 1)):
        wait_gathers(c, buf)
        fire_out(c, buf)
    for c, buf in ((NCHUNK - 2, 0), (NCHUNK - 1, 1)):
        wait_out(c, buf)


def kernel(q, table):
    q_rows = q.reshape(B // GSZ, GSZ).astype(jnp.int32)
    out = _sc_gather(q_rows, table)
    return out.reshape(q.shape[0], q.shape[1], D)


# speedup vs baseline: 9.2541x; 1.0077x over previous
import functools

import jax
import jax.numpy as jnp
from jax import lax
from jax.experimental import pallas as pl
from jax.experimental.pallas import tpu as pltpu
from jax.experimental.pallas import tpu_sc as plsc

D = 128
NC, NS = 2, 16
NW = NC * NS
B = 4096 * 200
BPW = B // NW
GSZ = 128
CH = 256
NSUB = CH // GSZ
NCHUNK = BPW // CH
NROWS_W = BPW // GSZ

_mesh = plsc.VectorSubcoreMesh(core_axis_name="c", subcore_axis_name="s")


@functools.partial(
    pl.kernel,
    out_type=jax.ShapeDtypeStruct((B, D), jnp.float32),
    mesh=_mesh,
    scratch_types=[
        pltpu.VMEM((NROWS_W, GSZ), jnp.int32),
        pltpu.VMEM((3, CH, D), jnp.float32),
        pltpu.SemaphoreType.DMA,
        pltpu.SemaphoreType.DMA,
        pltpu.SemaphoreType.DMA,
        pltpu.SemaphoreType.DMA,
        pltpu.SemaphoreType.DMA,
        pltpu.SemaphoreType.DMA,
    ],
)
def _sc_gather(q_hbm, table_hbm, out_hbm, idx_v, rows_v, g0, g1, g2, o0, o1, o2):
    wid = lax.axis_index("s") * NC + lax.axis_index("c")
    row0 = wid * NROWS_W
    base = wid * BPW
    gsem = (g0, g1, g2)
    osem = (o0, o1, o2)

    pltpu.sync_copy(q_hbm.at[pl.ds(row0, NROWS_W), :], idx_v)

    def fire_gathers(c, buf):
        for j in range(NSUB):
            pltpu.async_copy(
                table_hbm.at[idx_v.at[c * NSUB + j]],
                rows_v.at[buf, pl.ds(j * GSZ, GSZ), :],
                gsem[buf],
            )

    def wait_gathers(c, buf):
        for j in range(NSUB):
            pltpu.make_async_copy(
                table_hbm.at[idx_v.at[c * NSUB + j]],
                rows_v.at[buf, pl.ds(j * GSZ, GSZ), :],
                gsem[buf],
            ).wait()

    def fire_out(c, buf):
        pltpu.async_copy(
            rows_v.at[buf],
            out_hbm.at[pl.ds(base + c * CH, CH), :],
            osem[buf],
        )

    def wait_out(c, buf):
        pltpu.make_async_copy(
            rows_v.at[buf],
            out_hbm.at[pl.ds(base + c * CH, CH), :],
            osem[buf],
        ).wait()

    fire_gathers(0, 0)
    fire_gathers(1, 1)
    wait_gathers(0, 0)
    fire_out(0, 0)
    fire_gathers(2, 2)
    wait_gathers(1, 1)
    fire_out(1, 1)

    def step(c, buf):
        wait_gathers(c - 1, (buf + 2) % 3)
        fire_out(c - 1, (buf + 2) % 3)
        wait_out(c - 3, buf)
        fire_gathers(c, buf)

    def triple(p, carry):
        step(3 * p, 0)
        step(3 * p + 1, 1)
        step(3 * p + 2, 2)
        return carry

    lax.fori_loop(1, (NCHUNK - 1) // 3, triple, 0)

    step(NCHUNK - 1, 0)
    wait_gathers(NCHUNK - 1, 0)
    fire_out(NCHUNK - 1, 0)
    wait_out(NCHUNK - 3, 1)
    wait_out(NCHUNK - 2, 2)
    wait_out(NCHUNK - 1, 0)


def kernel(q, table):
    q_rows = q.reshape(B // GSZ, GSZ).astype(jnp.int32)
    out = _sc_gather(q_rows, table)
    return out.reshape(q.shape[0], q.shape[1], D)
